# baseline (speedup 1.0000x reference)
import functools

import jax
import jax.numpy as jnp
from jax import lax
from jax.experimental import pallas as pl
from jax.experimental.pallas import tpu as pltpu
from jax.experimental.pallas import tpu_sc as plsc

N_NODES = 10000
N_EDGES = 320000
FEAT = 128

NUM_SC = 2
NUM_SUBCORES = 16
NUM_WORKERS = NUM_SC * NUM_SUBCORES
CHUNK = 160
NCHUNKS = N_EDGES // CHUNK
K_MAX = (-(-NCHUNKS // NUM_WORKERS) + 3) // 4 * 4
ROWS_MAIN = 624


def _sc_segment_sum(x, eflat, zeros):
    mesh = plsc.VectorSubcoreMesh(core_axis_name="c", subcore_axis_name="s")

    @functools.partial(
        pl.kernel,
        out_type=jax.ShapeDtypeStruct((NUM_SC, N_NODES, FEAT), jnp.float32),
        mesh=mesh,
        scratch_types=(
            [pltpu.VMEM((CHUNK,), jnp.int32)] * 8
            + [pltpu.VMEM((CHUNK, FEAT), jnp.float32)] * 2
            + [pltpu.VMEM_SHARED((N_NODES, FEAT), jnp.float32)]
            + [pltpu.SemaphoreType.DMA] * 10
        ),
    )
    def agg_kernel(x_hbm, e_hbm, z_hbm, out_hbm, *scr):
        sidx = scr[0:4]
        didx = scr[4:8]
        rows = scr[8:10]
        acc = scr[10]
        semg = scr[11:13]
        semis = scr[13:17]
        semid = scr[17:21]
        cid = lax.axis_index("c")
        sid = lax.axis_index("s")

        wid = cid * NUM_SUBCORES + sid

        def valid(m):
            return wid + NUM_WORKERS * m < NCHUNKS

        def ebase(m):
            return (wid + NUM_WORKERS * m) * CHUNK

        def start_idx(m, j):
            @pl.when(valid(m))
            def _():
                base = ebase(m)
                pltpu.async_copy(e_hbm.at[pl.ds(base, CHUNK)], sidx[j],
                                 semis[j])
                pltpu.async_copy(e_hbm.at[pl.ds(N_EDGES + base, CHUNK)],
                                 didx[j], semid[j])

        def gather(m, j, rb):
            @pl.when(valid(m))
            def _():
                base = ebase(m)
                pltpu.make_async_copy(e_hbm.at[pl.ds(base, CHUNK)], sidx[j],
                                      semis[j]).wait()
                pltpu.async_copy(x_hbm.at[sidx[j]], rows[rb], semg[rb])

        def finish(m, j, rb):
            @pl.when(valid(m))
            def _():
                base = ebase(m)
                pltpu.make_async_copy(x_hbm.at[sidx[j]], rows[rb],
                                      semg[rb]).wait()
                pltpu.make_async_copy(e_hbm.at[pl.ds(N_EDGES + base, CHUNK)],
                                      didx[j], semid[j]).wait()
                pltpu.sync_copy(rows[rb], acc.at[didx[j]], add=True)

        for j in range(4):
            start_idx(j, j)

        r0 = sid * ROWS_MAIN
        rx = NUM_SUBCORES * ROWS_MAIN + sid * 8
        pltpu.sync_copy(z_hbm.at[pl.ds(r0, ROWS_MAIN)],
                        acc.at[pl.ds(r0, ROWS_MAIN)])

        @pl.when(sid < 2)
        def _():
            pltpu.sync_copy(z_hbm.at[pl.ds(rx, 8)], acc.at[pl.ds(rx, 8)])

        plsc.subcore_barrier()

        gather(0, 0, 0)
        gather(1, 1, 1)

        @pl.loop(0, K_MAX, step=4)
        def _(k):
            for m_off in range(4):
                m = k + m_off
                finish(m, m_off, m_off % 2)
                start_idx(m + 4, m_off)
                gather(m + 2, (m_off + 2) % 4, m_off % 2)

        plsc.subcore_barrier()
        pltpu.sync_copy(acc.at[pl.ds(r0, ROWS_MAIN)],
                        out_hbm.at[cid].at[pl.ds(r0, ROWS_MAIN)])

        @pl.when(sid < 2)
        def _():
            pltpu.sync_copy(acc.at[pl.ds(rx, 8)],
                            out_hbm.at[cid].at[pl.ds(rx, 8)])

    return agg_kernel(x, eflat, zeros)


def _mlp_block(x, p, W_a, b_a, W_b, b_b, eps, relu_last):
    n, f = x.shape
    out_f = W_b.shape[1]
    block = 10000

    def body(eps_ref, x_ref, p_ref, wa_ref, ba_ref, wb_ref, bb_ref,
             o_ref):
        a = x_ref[...] * (1.0 + eps_ref[0]) + p_ref[0] + p_ref[1]
        h = jnp.maximum(
            jnp.dot(a, wa_ref[...], preferred_element_type=jnp.float32)
            + ba_ref[...], 0.0)
        o = (jnp.dot(h, wb_ref[...], preferred_element_type=jnp.float32)
             + bb_ref[...])
        if relu_last:
            o = jnp.maximum(o, 0.0)
        o_ref[...] = o

    return pl.pallas_call(
        body,
        grid=(n // block,),
        in_specs=[
            pl.BlockSpec(memory_space=pltpu.SMEM),
            pl.BlockSpec((block, f), lambda i: (i, 0)),
            pl.BlockSpec((NUM_SC, block, f), lambda i: (0, i, 0)),
            pl.BlockSpec((f, W_a.shape[1]), lambda i: (0, 0)),
            pl.BlockSpec((1, W_a.shape[1]), lambda i: (0, 0)),
            pl.BlockSpec((W_b.shape[0], out_f), lambda i: (0, 0)),
            pl.BlockSpec((1, out_f), lambda i: (0, 0)),
        ],
        out_specs=pl.BlockSpec((block, out_f), lambda i: (i, 0)),
        out_shape=jax.ShapeDtypeStruct((n, out_f), jnp.float32),
    )(eps.reshape(1), x, p, W_a, b_a.reshape(1, -1), W_b,
      b_b.reshape(1, -1))


def kernel(x, edge_index, W1, b1, W2, b2, eps1, W3, b3, W4, b4, eps2):
    eflat = edge_index.astype(jnp.int32).reshape(2 * N_EDGES)
    zeros = jnp.zeros((N_NODES, FEAT), jnp.float32)

    p1 = _sc_segment_sum(x, eflat, zeros)
    h = _mlp_block(x, p1, W1, b1, W2, b2, eps1, relu_last=True)
    p2 = _sc_segment_sum(h, eflat, zeros)
    out = _mlp_block(h, p2, W3, b3, W4, b4, eps2, relu_last=False)
    return out


# speedup vs baseline: 1.1127x; 1.1127x over previous
import functools

import jax
import jax.numpy as jnp
from jax import lax
from jax.experimental import pallas as pl
from jax.experimental.pallas import tpu as pltpu
from jax.experimental.pallas import tpu_sc as plsc

N_NODES = 10000
N_EDGES = 320000
FEAT = 128

NUM_SC = 2
NUM_SUBCORES = 16
NUM_WORKERS = NUM_SC * NUM_SUBCORES
CHUNK = 128
NCHUNKS = N_EDGES // CHUNK
NROWS_BUF = 3
NIDX_BUF = 6
UNROLL = 6
K_MAX = (-(-NCHUNKS // NUM_WORKERS) + UNROLL - 1) // UNROLL * UNROLL
ROWS_MAIN = 624


def _sc_segment_sum(x, eflat, zeros):
    mesh = plsc.VectorSubcoreMesh(core_axis_name="c", subcore_axis_name="s")

    @functools.partial(
        pl.kernel,
        out_type=jax.ShapeDtypeStruct((NUM_SC, N_NODES, FEAT), jnp.float32),
        mesh=mesh,
        scratch_types=(
            [pltpu.VMEM((CHUNK,), jnp.int32)] * (2 * NIDX_BUF)
            + [pltpu.VMEM((CHUNK, FEAT), jnp.float32)] * NROWS_BUF
            + [pltpu.VMEM_SHARED((N_NODES, FEAT), jnp.float32)]
            + [pltpu.SemaphoreType.DMA] * (NROWS_BUF + 2 * NIDX_BUF)
        ),
    )
    def agg_kernel(x_hbm, e_hbm, z_hbm, out_hbm, *scr):
        sidx = scr[0:NIDX_BUF]
        didx = scr[NIDX_BUF:2 * NIDX_BUF]
        rows = scr[2 * NIDX_BUF:2 * NIDX_BUF + NROWS_BUF]
        acc = scr[2 * NIDX_BUF + NROWS_BUF]
        sems = scr[2 * NIDX_BUF + NROWS_BUF + 1:]
        semg = sems[0:NROWS_BUF]
        semis = sems[NROWS_BUF:NROWS_BUF + NIDX_BUF]
        semid = sems[NROWS_BUF + NIDX_BUF:]
        cid = lax.axis_index("c")
        sid = lax.axis_index("s")

        wid = cid * NUM_SUBCORES + sid

        def valid(m):
            return wid + NUM_WORKERS * m < NCHUNKS

        def ebase(m):
            return (wid + NUM_WORKERS * m) * CHUNK

        def start_idx(m, j):
            @pl.when(valid(m))
            def _():
                base = ebase(m)
                pltpu.async_copy(e_hbm.at[pl.ds(base, CHUNK)], sidx[j],
                                 semis[j])
                pltpu.async_copy(e_hbm.at[pl.ds(N_EDGES + base, CHUNK)],
                                 didx[j], semid[j])

        def gather(m, j, rb):
            @pl.when(valid(m))
            def _():
                base = ebase(m)
                pltpu.make_async_copy(e_hbm.at[pl.ds(base, CHUNK)], sidx[j],
                                      semis[j]).wait()
                pltpu.async_copy(x_hbm.at[sidx[j]], rows[rb], semg[rb])

        def finish(m, j, rb):
            @pl.when(valid(m))
            def _():
                base = ebase(m)
                pltpu.make_async_copy(x_hbm.at[sidx[j]], rows[rb],
                                      semg[rb]).wait()
                pltpu.make_async_copy(e_hbm.at[pl.ds(N_EDGES + base, CHUNK)],
                                      didx[j], semid[j]).wait()
                pltpu.sync_copy(rows[rb], acc.at[didx[j]], add=True)

        for j in range(NIDX_BUF):
            start_idx(j, j)

        r0 = sid * ROWS_MAIN
        rx = NUM_SUBCORES * ROWS_MAIN + sid * 8
        pltpu.sync_copy(z_hbm.at[pl.ds(r0, ROWS_MAIN)],
                        acc.at[pl.ds(r0, ROWS_MAIN)])

        @pl.when(sid < 2)
        def _():
            pltpu.sync_copy(z_hbm.at[pl.ds(rx, 8)], acc.at[pl.ds(rx, 8)])

        plsc.subcore_barrier()

        gather(0, 0, 0)
        gather(1, 1, 1)
        gather(2, 2, 2)

        @pl.loop(0, K_MAX, step=UNROLL)
        def _(k):
            for i in range(UNROLL):
                m = k + i
                finish(m, i % NIDX_BUF, i % NROWS_BUF)
                start_idx(m + NIDX_BUF, i % NIDX_BUF)
                gather(m + NROWS_BUF, (i + NROWS_BUF) % NIDX_BUF,
                       i % NROWS_BUF)

        plsc.subcore_barrier()
        pltpu.sync_copy(acc.at[pl.ds(r0, ROWS_MAIN)],
                        out_hbm.at[cid].at[pl.ds(r0, ROWS_MAIN)])

        @pl.when(sid < 2)
        def _():
            pltpu.sync_copy(acc.at[pl.ds(rx, 8)],
                            out_hbm.at[cid].at[pl.ds(rx, 8)])

    return agg_kernel(x, eflat, zeros)


def _mlp_block(x, p, W_a, b_a, W_b, b_b, eps, relu_last):
    n, f = x.shape
    out_f = W_b.shape[1]
    block = 5000

    def body(eps_ref, x_ref, p_ref, wa_ref, ba_ref, wb_ref, bb_ref,
             o_ref):
        a = x_ref[...] * (1.0 + eps_ref[0]) + p_ref[0] + p_ref[1]
        h = jnp.maximum(
            jnp.dot(a, wa_ref[...], preferred_element_type=jnp.float32)
            + ba_ref[...], 0.0)
        o = (jnp.dot(h, wb_ref[...], preferred_element_type=jnp.float32)
             + bb_ref[...])
        if relu_last:
            o = jnp.maximum(o, 0.0)
        o_ref[...] = o

    return pl.pallas_call(
        body,
        grid=(n // block,),
        in_specs=[
            pl.BlockSpec(memory_space=pltpu.SMEM),
            pl.BlockSpec((block, f), lambda i: (i, 0)),
            pl.BlockSpec((NUM_SC, block, f), lambda i: (0, i, 0)),
            pl.BlockSpec((f, W_a.shape[1]), lambda i: (0, 0)),
            pl.BlockSpec((1, W_a.shape[1]), lambda i: (0, 0)),
            pl.BlockSpec((W_b.shape[0], out_f), lambda i: (0, 0)),
            pl.BlockSpec((1, out_f), lambda i: (0, 0)),
        ],
        out_specs=pl.BlockSpec((block, out_f), lambda i: (i, 0)),
        out_shape=jax.ShapeDtypeStruct((n, out_f), jnp.float32),
    )(eps.reshape(1), x, p, W_a, b_a.reshape(1, -1), W_b,
      b_b.reshape(1, -1))


def kernel(x, edge_index, W1, b1, W2, b2, eps1, W3, b3, W4, b4, eps2):
    eflat = edge_index.astype(jnp.int32).reshape(2 * N_EDGES)
    zeros = jnp.zeros((N_NODES, FEAT), jnp.float32)

    p1 = _sc_segment_sum(x, eflat, zeros)
    h = _mlp_block(x, p1, W1, b1, W2, b2, eps1, relu_last=True)
    p2 = _sc_segment_sum(h, eflat, zeros)
    out = _mlp_block(h, p2, W3, b3, W4, b4, eps2, relu_last=False)
    return out
